# baseline (speedup 1.0000x reference)
import functools

import jax
import jax.numpy as jnp
from jax import lax
from jax.experimental import pallas as pl
from jax.experimental.pallas import tpu as pltpu
from jax.experimental.pallas import tpu_sc as plsc

NUM_CORES = 2
NUM_SUBCORES = 16
NUM_WORKERS = NUM_CORES * NUM_SUBCORES
LANES = 16

BATCH = 16384
RANK = 16
ROWS_PER_GRANULE = 128 // RANK

BPW = BATCH // NUM_WORKERS
CHUNK = 128
NCHUNK = BPW // CHUNK
NGROUP = CHUNK // LANES


def _bpr_body(uid_hbm, pid_hbm, nid_hbm, utab, itab,
              pos_hbm, neg_hbm,
              uid_v, pid_v, nid_v, ugid, pgid, ngid,
              ubuf, pbuf, nbuf, pos_v, neg_v, sem):
    c = lax.axis_index("c")
    s = lax.axis_index("s")
    wid = s * NUM_CORES + c
    base = wid * BPW

    pltpu.sync_copy(uid_hbm.at[pl.ds(base, BPW)], uid_v)
    pltpu.sync_copy(pid_hbm.at[pl.ds(base, BPW)], pid_v)
    pltpu.sync_copy(nid_hbm.at[pl.ds(base, BPW)], nid_v)

    def gids(i, carry):
        sl = pl.ds(i * LANES, LANES)
        ugid[sl] = lax.shift_right_logical(uid_v[sl], 3)
        pgid[sl] = lax.shift_right_logical(pid_v[sl], 3)
        ngid[sl] = lax.shift_right_logical(nid_v[sl], 3)
        return carry

    lax.fori_loop(0, BPW // LANES, gids, 0)

    for ch in range(NCHUNK):
        csl = pl.ds(ch * CHUNK, CHUNK)
        hu = pltpu.make_async_copy(utab.at[ugid.at[csl]], ubuf, sem)
        hp = pltpu.make_async_copy(itab.at[pgid.at[csl]], pbuf, sem)
        hn = pltpu.make_async_copy(itab.at[ngid.at[csl]], nbuf, sem)
        hu.start()
        hp.start()
        hn.start()
        hu.wait()
        hp.wait()
        hn.wait()

        def group(g, carry):
            gsl = pl.ds(ch * CHUNK + g * LANES, LANES)
            rows = g * LANES + lax.iota(jnp.int32, LANES)
            ucol = (uid_v[gsl] & (ROWS_PER_GRANULE - 1)) * RANK
            pcol = (pid_v[gsl] & (ROWS_PER_GRANULE - 1)) * RANK
            ncol = (nid_v[gsl] & (ROWS_PER_GRANULE - 1)) * RANK
            accp = jnp.zeros((LANES,), jnp.float32)
            accn = jnp.zeros((LANES,), jnp.float32)
            for f in range(RANK):
                u = plsc.load_gather(ubuf, [rows, ucol + f])
                p = plsc.load_gather(pbuf, [rows, pcol + f])
                n = plsc.load_gather(nbuf, [rows, ncol + f])
                accp = accp + u * p
                accn = accn + u * n
            pos_v[gsl] = accp
            neg_v[gsl] = accn
            return carry

        lax.fori_loop(0, NGROUP, group, 0)

    pltpu.sync_copy(pos_v, pos_hbm.at[pl.ds(base, BPW)])
    pltpu.sync_copy(neg_v, neg_hbm.at[pl.ds(base, BPW)])


@functools.partial(
    pl.kernel,
    out_type=(jax.ShapeDtypeStruct((BATCH,), jnp.float32),
              jax.ShapeDtypeStruct((BATCH,), jnp.float32)),
    mesh=plsc.VectorSubcoreMesh(core_axis_name="c", subcore_axis_name="s"),
    scratch_types=[
        pltpu.VMEM((BPW,), jnp.int32),
        pltpu.VMEM((BPW,), jnp.int32),
        pltpu.VMEM((BPW,), jnp.int32),
        pltpu.VMEM((BPW,), jnp.int32),
        pltpu.VMEM((BPW,), jnp.int32),
        pltpu.VMEM((BPW,), jnp.int32),
        pltpu.VMEM((CHUNK, 128), jnp.float32),
        pltpu.VMEM((CHUNK, 128), jnp.float32),
        pltpu.VMEM((CHUNK, 128), jnp.float32),
        pltpu.VMEM((BPW,), jnp.float32),
        pltpu.VMEM((BPW,), jnp.float32),
        pltpu.SemaphoreType.DMA,
    ],
    compiler_params=pltpu.CompilerParams(needs_layout_passes=False,
                                         use_tc_tiling_on_sc=False),
)
def _bpr_sc(uid, pid, nid, utab, itab, pos_out, neg_out, *scratch):
    _bpr_body(uid, pid, nid, utab, itab, pos_out, neg_out, *scratch)


def kernel(user_ids, pos_items, neg_items, user_table, item_table):
    n_gran = user_table.shape[0] * RANK // 128
    return _bpr_sc(user_ids.astype(jnp.int32),
                   pos_items.astype(jnp.int32),
                   neg_items.astype(jnp.int32),
                   user_table.reshape(n_gran, 128),
                   item_table.reshape(n_gran, 128))
